# baseline (speedup 1.0000x reference)
import functools

import jax
import jax.numpy as jnp
from jax import lax
from jax.experimental import pallas as pl
from jax.experimental.pallas import tpu as pltpu
from jax.experimental.pallas import tpu_sc as plsc

NC = 2
NS = 16
CH = 128
KJ = 8

N_NODES = 100000
ACC_ROWS = N_NODES + 96
ROWS_PER_TILE = ACC_ROWS // NS
ZR = 512


def _pad_idx(idx, total, fill):
    pad = total + KJ * CH - idx.shape[0]
    out = jnp.concatenate([idx, jnp.full((pad,), fill, jnp.int32)])
    return out.reshape(out.shape[0] // CH, CH)


def _ceil_to(x, m):
    return ((x + m - 1) // m) * m


def _zero_acc(zero_h, acc, s):
    r0 = s * ROWS_PER_TILE
    nfull = ROWS_PER_TILE // ZR
    rem = ROWS_PER_TILE - nfull * ZR
    for z in range(nfull):
        pltpu.sync_copy(zero_h, acc.at[pl.ds(r0 + z * ZR, ZR)])
    if rem:
        pltpu.sync_copy(zero_h.at[pl.ds(0, rem)],
                        acc.at[pl.ds(r0 + nfull * ZR, rem)])


def _counts_call(dst2d, int2d, bnd2d, basis, zeros):
    td = (dst2d.shape[0] - KJ) // (NC * NS)
    ti = (int2d.shape[0] - KJ) // (NC * NS)
    tb = (bnd2d.shape[0] - KJ) // (NC * NS)
    mesh = plsc.VectorSubcoreMesh(core_axis_name="c", subcore_axis_name="s")

    @functools.partial(
        pl.kernel,
        out_type=jax.ShapeDtypeStruct((NC, ACC_ROWS, 16), jnp.float32),
        mesh=mesh,
        compiler_params=pltpu.CompilerParams(use_tc_tiling_on_sc=False),
        scratch_types=[
            pltpu.VMEM_SHARED((ACC_ROWS, 16), jnp.float32),
            pltpu.VMEM((KJ, CH), jnp.int32),
            pltpu.VMEM((KJ, CH), jnp.int32),
            pltpu.VMEM((3, CH, 16), jnp.float32),
            pltpu.SemaphoreType.DMA,
            pltpu.SemaphoreType.DMA,
        ],
    )
    def k(dst_h, int_h, bnd_h, basis_h, zero_h, out_h,
          acc, ibufa, ibufb, bbuf, isem, ssem):
        c = lax.axis_index("c")
        s = lax.axis_index("s")
        w = c * NS + s

        pltpu.sync_copy(basis_h, bbuf)
        _zero_acc(zero_h, acc, s)
        plsc.subcore_barrier()

        def scan_list(idx_h, rows_per_tile, which):
            base = w * rows_per_tile
            nblk = rows_per_tile // KJ

            def drain():
                for j in range(KJ):
                    pltpu.make_async_copy(
                        bbuf.at[which], acc.at[ibufa.at[j]], ssem).wait()

            def blk(i, _):
                @pl.when(i > 0)
                def _():
                    drain()

                pltpu.sync_copy(idx_h.at[pl.ds(base + i * KJ, KJ)], ibufa)
                for j in range(KJ):
                    pltpu.async_copy(bbuf.at[which], acc.at[ibufa.at[j]],
                                     ssem, add=True)
                return 0

            lax.fori_loop(0, nblk, blk, 0)
            drain()

        scan_list(dst_h, td, 0)
        scan_list(int_h, ti, 1)
        scan_list(bnd_h, tb, 2)

        plsc.subcore_barrier()
        r0 = s * ROWS_PER_TILE
        pltpu.sync_copy(acc.at[pl.ds(r0, ROWS_PER_TILE)],
                        out_h.at[c].at[pl.ds(r0, ROWS_PER_TILE)])

    return k(dst2d, int2d, bnd2d, basis, zeros)


def _scatter_call(src2d, dst2d, gtab, zeros):
    tpt = (src2d.shape[0] - KJ) // NS
    mesh = plsc.VectorSubcoreMesh(core_axis_name="c", subcore_axis_name="s")

    @functools.partial(
        pl.kernel,
        out_type=jax.ShapeDtypeStruct((NC, ACC_ROWS, 16), jnp.float32),
        mesh=mesh,
        compiler_params=pltpu.CompilerParams(use_tc_tiling_on_sc=False),
        scratch_types=[
            pltpu.VMEM_SHARED((ACC_ROWS, 16), jnp.float32),
            pltpu.VMEM((KJ, CH), jnp.int32),
            pltpu.VMEM((KJ, CH), jnp.int32),
            pltpu.VMEM((KJ, CH), jnp.int32),
            pltpu.VMEM((KJ, CH), jnp.int32),
            pltpu.VMEM((KJ, CH, 16), jnp.float32),
            pltpu.SemaphoreType.DMA,
            pltpu.SemaphoreType.DMA,
            pltpu.SemaphoreType.DMA,
        ],
    )
    def k(src_h, dst_h, gtab_h, zero_h, out_h,
          acc, sidxa, sidxb, didxa, didxb, rows, isem, gsem, ssem):
        c = lax.axis_index("c")
        s = lax.axis_index("s")

        _zero_acc(zero_h, acc, s)
        plsc.subcore_barrier()

        base = s * tpt
        nblk = tpt // KJ

        def drain_scatters():
            for j in range(KJ):
                pltpu.make_async_copy(
                    rows.at[j], acc.at[didxa.at[j]], ssem).wait()

        def blk(i, _):
            @pl.when(i > 0)
            def _():
                drain_scatters()

            off = base + i * KJ
            pltpu.sync_copy(src_h.at[pl.ds(off, KJ)], sidxa)
            pltpu.sync_copy(dst_h.at[pl.ds(off, KJ)], didxa)
            descs = []
            for j in range(KJ):
                descs.append(
                    pltpu.async_copy(gtab_h.at[c].at[sidxa.at[j]],
                                     rows.at[j], gsem))
            for j in range(KJ):
                descs[j].wait()
                pltpu.async_copy(rows.at[j], acc.at[didxa.at[j]],
                                 ssem, add=True)
            return 0

        lax.fori_loop(0, nblk, blk, 0)
        drain_scatters()

        plsc.subcore_barrier()
        r0 = s * ROWS_PER_TILE
        pltpu.sync_copy(acc.at[pl.ds(r0, ROWS_PER_TILE)],
                        out_h.at[c].at[pl.ds(r0, ROWS_PER_TILE)])

    return k(src2d, dst2d, gtab, zeros)


BR = 2000


def _full(shape):
    return pl.BlockSpec(shape, lambda i: (0,) * len(shape))


def _plane(p):
    return pl.BlockSpec((1, BR, 16), lambda i, p=p: (p, i, 0))


def _planes2():
    return pl.BlockSpec((2, BR, 16), lambda i: (0, i, 0))


def _rows(width):
    return pl.BlockSpec((BR, width), lambda i: (i, 0))


def _tc_entry_body(in8_r, c0_r, c1_r, wb1_r, bb1_r, wb2_r, bb2_r,
                   wi1_r, bi1_r, wi2_r, bi2_r, wc1_r,
                   g_r, dv_r):
    xin = in8_r[...]
    cnt = c0_r[0] + c1_r[0]
    dinv = lax.rsqrt(1.0 + cnt[:, 0:1])
    mi = cnt[:, 1:2] > 0.0
    mb = cnt[:, 2:3] > 0.0
    tb = jnp.maximum(xin @ wb1_r[...] + bb1_r[...], 0.0) @ wb2_r[...] + bb2_r[...]
    ti = jnp.maximum(xin @ wi1_r[...] + bi1_r[...], 0.0) @ wi2_r[...] + bi2_r[...]
    h0 = jnp.maximum(jnp.where(mi, ti, jnp.where(mb, tb, 0.0)), 0.0)
    g = (h0 @ wc1_r[...]) * dinv
    g_r[0] = g[:, :16]
    g_r[1] = g[:, 16:]
    dv_r[...] = dinv


def _tc_entry(in8, cnt, wb1, bb1, wb2, bb2, wi1, bi1, wi2, bi2, wc1):
    n = in8.shape[0]
    return pl.pallas_call(
        _tc_entry_body,
        grid=(n // BR,),
        in_specs=[_rows(8), _plane(0), _plane(1),
                  _full((8, 32)), _full((1, 32)), _full((32, 32)), _full((1, 32)),
                  _full((8, 32)), _full((1, 32)), _full((32, 32)), _full((1, 32)),
                  _full((32, 32))],
        out_specs=[_planes2(), _rows(1)],
        out_shape=[jax.ShapeDtypeStruct((2, n, 16), jnp.float32),
                   jax.ShapeDtypeStruct((n, 1), jnp.float32)],
    )(in8, cnt, cnt, wb1, bb1, wb2, bb2, wi1, bi1, wi2, bi2, wc1)


def _tc_mid_body(s0_r, s1_r, g_r, dv_r, b_r, wn_r, o_r):
    dinv = dv_r[...]
    sagg = jnp.concatenate([s0_r[0], s1_r[0]], axis=1)
    g = jnp.concatenate([g_r[0], g_r[1]], axis=1)
    h = jnp.maximum(dinv * (sagg + g) + b_r[...], 0.0)
    gn = (h @ wn_r[...]) * dinv
    o_r[0] = gn[:, :16]
    o_r[1] = gn[:, 16:]


def _tc_mid(s_acc, g, dv, b, wn):
    n = g.shape[1]
    return pl.pallas_call(
        _tc_mid_body,
        grid=(n // BR,),
        in_specs=[_plane(0), _plane(1), _planes2(), _rows(1),
                  _full((1, 32)), _full((32, 32))],
        out_specs=_planes2(),
        out_shape=jax.ShapeDtypeStruct((2, n, 16), jnp.float32),
    )(s_acc, s_acc, g, dv, b, wn)


def _tc_final_body(s0_r, s1_r, g_r, dv_r, b_r, wfc_r, bfc_r, o_r):
    dinv = dv_r[...]
    sagg = jnp.concatenate([s0_r[0], s1_r[0]], axis=1)
    g = jnp.concatenate([g_r[0], g_r[1]], axis=1)
    h = jnp.maximum(dinv * (sagg + g) + b_r[...], 0.0)
    o_r[...] = h @ wfc_r[...] + bfc_r[...]


def _tc_final(s_acc, g, dv, b, wfc, bfc):
    n = g.shape[1]
    return pl.pallas_call(
        _tc_final_body,
        grid=(n // BR,),
        in_specs=[_plane(0), _plane(1), _planes2(), _rows(1),
                  _full((1, 32)), _full((32, 1)), _full((1, 1))],
        out_specs=_rows(1),
        out_shape=jax.ShapeDtypeStruct((n, 1), jnp.float32),
    )(s_acc, s_acc, g, dv, b, wfc, bfc)


def kernel(x, y, edge_index, boundary_index, interior_index,
           W_b1, b_b1, W_b2, b_b2, W_i1, b_i1, W_i2, b_i2,
           Wc1, bc1, Wc2, bc2, Wc3, bc3, W_fc, b_fc):
    n = x.shape[0]
    e = edge_index.shape[1]
    src, dst = edge_index[0], edge_index[1]

    blkc = CH * KJ
    ep = _ceil_to(e, NC * NS * blkc * 2)
    src_p = _pad_idx(src, ep, 0)
    dst_p = _pad_idx(dst, ep, n)
    int_p = _pad_idx(interior_index,
                     _ceil_to(interior_index.shape[0], NC * NS * blkc), n)
    bnd_p = _pad_idx(boundary_index,
                     _ceil_to(boundary_index.shape[0], NC * NS * blkc), n)

    basis = jnp.zeros((3, CH, 16), jnp.float32)
    basis = basis.at[0, :, 0].set(1.0).at[1, :, 1].set(1.0).at[2, :, 2].set(1.0)
    zeros = jnp.zeros((ZR, 16), jnp.float32)

    wb1 = jnp.pad(W_b1, ((0, 0), (0, 5))).T
    wi1 = jnp.pad(W_i1, ((0, 0), (0, 6))).T
    in8 = jnp.concatenate([x, y[:, None], jnp.zeros((n, 5), jnp.float32)],
                          axis=1)
    wc = [Wc1.T, Wc2.T, Wc3.T]
    bc = [bc1[None, :], bc2[None, :], bc3[None, :]]

    cnt = _counts_call(dst_p, int_p, bnd_p, basis, zeros)

    g, dv = _tc_entry(in8, cnt, wb1, b_b1[None, :], W_b2.T, b_b2[None, :],
                      wi1, b_i1[None, :], W_i2.T, b_i2[None, :], wc[0])

    for layer in range(3):
        s_acc = _scatter_call(src_p, dst_p, g, zeros)
        if layer < 2:
            g = _tc_mid(s_acc, g, dv, bc[layer], wc[layer + 1])
        else:
            out = _tc_final(s_acc, g, dv, bc[layer], W_fc.T, b_fc[None, :])
    return out


# speedup vs baseline: 1.3124x; 1.3124x over previous
import functools

import jax
import jax.numpy as jnp
from jax import lax
from jax.experimental import pallas as pl
from jax.experimental.pallas import tpu as pltpu
from jax.experimental.pallas import tpu_sc as plsc

NC = 2
NS = 16
CH = 128
KJ = 8

N_NODES = 100000
ACC_ROWS = N_NODES + 96
ROWS_PER_TILE = ACC_ROWS // NS
ZR = 512


def _pad_idx(idx, total, fill):
    pad = total - idx.shape[0]
    out = jnp.concatenate([idx, jnp.full((pad,), fill, jnp.int32)])
    return out.reshape(out.shape[0] // CH, CH)


def _ceil_to(x, m):
    return ((x + m - 1) // m) * m


def _zero_acc(zero_h, acc, s):
    r0 = s * ROWS_PER_TILE
    nfull = ROWS_PER_TILE // ZR
    rem = ROWS_PER_TILE - nfull * ZR
    for z in range(nfull):
        pltpu.sync_copy(zero_h, acc.at[pl.ds(r0 + z * ZR, ZR)])
    if rem:
        pltpu.sync_copy(zero_h.at[pl.ds(0, rem)],
                        acc.at[pl.ds(r0 + nfull * ZR, rem)])


def _counts_call(dst2d, int2d, bnd2d, basis, zeros):
    td = dst2d.shape[0] // (NC * NS)
    ti = int2d.shape[0] // (NC * NS)
    tb = bnd2d.shape[0] // (NC * NS)
    mesh = plsc.VectorSubcoreMesh(core_axis_name="c", subcore_axis_name="s")

    @functools.partial(
        pl.kernel,
        out_type=jax.ShapeDtypeStruct((NC, ACC_ROWS, 16), jnp.float32),
        mesh=mesh,
        compiler_params=pltpu.CompilerParams(use_tc_tiling_on_sc=False),
        scratch_types=[
            pltpu.VMEM_SHARED((ACC_ROWS, 16), jnp.float32),
            pltpu.VMEM((KJ, CH), jnp.int32),
            pltpu.VMEM((3, CH, 16), jnp.float32),
            pltpu.SemaphoreType.DMA,
        ],
    )
    def k(dst_h, int_h, bnd_h, basis_h, zero_h, out_h, acc, ibufa, bbuf, ssem):
        c = lax.axis_index("c")
        s = lax.axis_index("s")
        w = c * NS + s

        pltpu.sync_copy(basis_h, bbuf)
        _zero_acc(zero_h, acc, s)
        plsc.subcore_barrier()

        def scan_list(idx_h, rows_per_tile, which):
            base = w * rows_per_tile
            nblk = rows_per_tile // KJ

            def drain():
                for j in range(KJ):
                    pltpu.make_async_copy(
                        bbuf.at[which], acc.at[ibufa.at[j]], ssem).wait()

            def blk(i, _):
                @pl.when(i > 0)
                def _():
                    drain()

                pltpu.sync_copy(idx_h.at[pl.ds(base + i * KJ, KJ)], ibufa)
                for j in range(KJ):
                    pltpu.async_copy(bbuf.at[which], acc.at[ibufa.at[j]],
                                     ssem, add=True)
                return 0

            lax.fori_loop(0, nblk, blk, 0)
            drain()

        scan_list(dst_h, td, 0)
        scan_list(int_h, ti, 1)
        scan_list(bnd_h, tb, 2)

        plsc.subcore_barrier()
        r0 = s * ROWS_PER_TILE
        pltpu.sync_copy(acc.at[pl.ds(r0, ROWS_PER_TILE)],
                        out_h.at[c].at[pl.ds(r0, ROWS_PER_TILE)])

    return k(dst2d, int2d, bnd2d, basis, zeros)


def _scatter_call(src2d, dst2d, gtab, zeros):
    tpt = src2d.shape[0] // NS
    mesh = plsc.VectorSubcoreMesh(core_axis_name="c", subcore_axis_name="s")

    @functools.partial(
        pl.kernel,
        out_type=jax.ShapeDtypeStruct((NC, ACC_ROWS, 16), jnp.float32),
        mesh=mesh,
        compiler_params=pltpu.CompilerParams(use_tc_tiling_on_sc=False),
        scratch_types=[
            pltpu.VMEM_SHARED((ACC_ROWS, 16), jnp.float32),
            pltpu.VMEM((KJ, CH), jnp.int32),
            pltpu.VMEM((KJ, CH), jnp.int32),
            pltpu.VMEM((KJ, CH, 16), jnp.float32),
            pltpu.SemaphoreType.DMA,
            pltpu.SemaphoreType.DMA,
        ],
    )
    def k(src_h, dst_h, gtab_h, zero_h, out_h,
          acc, sidxa, didxa, rows, gsem, ssem):
        c = lax.axis_index("c")
        s = lax.axis_index("s")

        _zero_acc(zero_h, acc, s)
        plsc.subcore_barrier()

        base = s * tpt
        nblk = tpt // KJ

        def drain_scatters():
            for j in range(KJ):
                pltpu.make_async_copy(
                    rows.at[j], acc.at[didxa.at[j]], ssem).wait()

        def blk(i, _):
            @pl.when(i > 0)
            def _():
                drain_scatters()

            off = base + i * KJ
            pltpu.sync_copy(src_h.at[pl.ds(off, KJ)], sidxa)
            pltpu.sync_copy(dst_h.at[pl.ds(off, KJ)], didxa)
            descs = []
            for j in range(KJ):
                descs.append(
                    pltpu.async_copy(gtab_h.at[c].at[sidxa.at[j]],
                                     rows.at[j], gsem))
            for j in range(KJ):
                descs[j].wait()
                pltpu.async_copy(rows.at[j], acc.at[didxa.at[j]],
                                 ssem, add=True)
            return 0

        lax.fori_loop(0, nblk, blk, 0)
        drain_scatters()

        plsc.subcore_barrier()
        r0 = s * ROWS_PER_TILE
        pltpu.sync_copy(acc.at[pl.ds(r0, ROWS_PER_TILE)],
                        out_h.at[c].at[pl.ds(r0, ROWS_PER_TILE)])

    return k(src2d, dst2d, gtab, zeros)


BR = 2000


def _full(shape):
    return pl.BlockSpec(shape, lambda i: (0,) * len(shape))


def _plane(p):
    return pl.BlockSpec((1, BR, 16), lambda i, p=p: (p, i, 0))


def _planes2():
    return pl.BlockSpec((2, BR, 16), lambda i: (0, i, 0))


def _rows(width):
    return pl.BlockSpec((BR, width), lambda i: (i, 0))


def _tc_entry_body(in8_r, c0_r, c1_r, wb1_r, bb1_r, wb2_r, bb2_r,
                   wi1_r, bi1_r, wi2_r, bi2_r, wc1_r,
                   g_r, dv_r):
    xin = in8_r[...]
    cnt = c0_r[0] + c1_r[0]
    dinv = lax.rsqrt(1.0 + cnt[:, 0:1])
    mi = cnt[:, 1:2] > 0.0
    mb = cnt[:, 2:3] > 0.0
    tb = jnp.maximum(xin @ wb1_r[...] + bb1_r[...], 0.0) @ wb2_r[...] + bb2_r[...]
    ti = jnp.maximum(xin @ wi1_r[...] + bi1_r[...], 0.0) @ wi2_r[...] + bi2_r[...]
    h0 = jnp.maximum(jnp.where(mi, ti, jnp.where(mb, tb, 0.0)), 0.0)
    g = (h0 @ wc1_r[...]) * dinv
    g_r[0] = g[:, :16]
    g_r[1] = g[:, 16:]
    dv_r[...] = dinv


def _tc_entry(in8, cnt, wb1, bb1, wb2, bb2, wi1, bi1, wi2, bi2, wc1):
    n = in8.shape[0]
    return pl.pallas_call(
        _tc_entry_body,
        grid=(n // BR,),
        in_specs=[_rows(8), _plane(0), _plane(1),
                  _full((8, 32)), _full((1, 32)), _full((32, 32)), _full((1, 32)),
                  _full((8, 32)), _full((1, 32)), _full((32, 32)), _full((1, 32)),
                  _full((32, 32))],
        out_specs=[_planes2(), _rows(1)],
        out_shape=[jax.ShapeDtypeStruct((2, n, 16), jnp.float32),
                   jax.ShapeDtypeStruct((n, 1), jnp.float32)],
    )(in8, cnt, cnt, wb1, bb1, wb2, bb2, wi1, bi1, wi2, bi2, wc1)


def _tc_mid_body(s0_r, s1_r, g_r, dv_r, b_r, wn_r, o_r):
    dinv = dv_r[...]
    sagg = jnp.concatenate([s0_r[0], s1_r[0]], axis=1)
    g = jnp.concatenate([g_r[0], g_r[1]], axis=1)
    h = jnp.maximum(dinv * (sagg + g) + b_r[...], 0.0)
    gn = (h @ wn_r[...]) * dinv
    o_r[0] = gn[:, :16]
    o_r[1] = gn[:, 16:]


def _tc_mid(s_acc, g, dv, b, wn):
    n = g.shape[1]
    return pl.pallas_call(
        _tc_mid_body,
        grid=(n // BR,),
        in_specs=[_plane(0), _plane(1), _planes2(), _rows(1),
                  _full((1, 32)), _full((32, 32))],
        out_specs=_planes2(),
        out_shape=jax.ShapeDtypeStruct((2, n, 16), jnp.float32),
    )(s_acc, s_acc, g, dv, b, wn)


def _tc_final_body(s0_r, s1_r, g_r, dv_r, b_r, wfc_r, bfc_r, o_r):
    dinv = dv_r[...]
    sagg = jnp.concatenate([s0_r[0], s1_r[0]], axis=1)
    g = jnp.concatenate([g_r[0], g_r[1]], axis=1)
    h = jnp.maximum(dinv * (sagg + g) + b_r[...], 0.0)
    o_r[...] = h @ wfc_r[...] + bfc_r[...]


def _tc_final(s_acc, g, dv, b, wfc, bfc):
    n = g.shape[1]
    return pl.pallas_call(
        _tc_final_body,
        grid=(n // BR,),
        in_specs=[_plane(0), _plane(1), _planes2(), _rows(1),
                  _full((1, 32)), _full((32, 1)), _full((1, 1))],
        out_specs=_rows(1),
        out_shape=jax.ShapeDtypeStruct((n, 1), jnp.float32),
    )(s_acc, s_acc, g, dv, b, wfc, bfc)


def kernel(x, y, edge_index, boundary_index, interior_index,
           W_b1, b_b1, W_b2, b_b2, W_i1, b_i1, W_i2, b_i2,
           Wc1, bc1, Wc2, bc2, Wc3, bc3, W_fc, b_fc):
    n = x.shape[0]
    e = edge_index.shape[1]
    src, dst = edge_index[0], edge_index[1]

    blkc = CH * KJ
    ep = _ceil_to(e, NS * blkc)
    ep = _ceil_to(ep, NC * NS * blkc)
    src_p = _pad_idx(src, ep, 0)
    dst_p = _pad_idx(dst, ep, n)
    int_p = _pad_idx(interior_index,
                     _ceil_to(interior_index.shape[0], NC * NS * blkc), n)
    bnd_p = _pad_idx(boundary_index,
                     _ceil_to(boundary_index.shape[0], NC * NS * blkc), n)

    basis = jnp.zeros((3, CH, 16), jnp.float32)
    basis = basis.at[0, :, 0].set(1.0).at[1, :, 1].set(1.0).at[2, :, 2].set(1.0)
    zeros = jnp.zeros((ZR, 16), jnp.float32)

    wb1 = jnp.pad(W_b1, ((0, 0), (0, 5))).T
    wi1 = jnp.pad(W_i1, ((0, 0), (0, 6))).T
    in8 = jnp.concatenate([x, y[:, None], jnp.zeros((n, 5), jnp.float32)],
                          axis=1)
    wc = [Wc1.T, Wc2.T, Wc3.T]
    bc = [bc1[None, :], bc2[None, :], bc3[None, :]]

    cnt = _counts_call(dst_p, int_p, bnd_p, basis, zeros)

    g, dv = _tc_entry(in8, cnt, wb1, b_b1[None, :], W_b2.T, b_b2[None, :],
                      wi1, b_i1[None, :], W_i2.T, b_i2[None, :], wc[0])

    for layer in range(3):
        s_acc = _scatter_call(src_p, dst_p, g, zeros)
        if layer < 2:
            g = _tc_mid(s_acc, g, dv, bc[layer], wc[layer + 1])
        else:
            out = _tc_final(s_acc, g, dv, bc[layer], W_fc.T, b_fc[None, :])
    return out


# speedup vs baseline: 1.3948x; 1.0628x over previous
import functools

import jax
import jax.numpy as jnp
from jax import lax
from jax.experimental import pallas as pl
from jax.experimental.pallas import tpu as pltpu
from jax.experimental.pallas import tpu_sc as plsc

NC = 2
NS = 16
CH = 128
KJ = 8

N_NODES = 100000
ACC_ROWS = N_NODES + 96
ROWS_PER_TILE = ACC_ROWS // NS
ZR = 512


def _pad_idx(idx, total, fill):
    pad = total - idx.shape[0]
    out = jnp.concatenate([idx, jnp.full((pad,), fill, jnp.int32)])
    return out.reshape(out.shape[0] // CH, CH)


def _ceil_to(x, m):
    return ((x + m - 1) // m) * m


def _zero_acc(zero_h, acc, s):
    r0 = s * ROWS_PER_TILE
    nfull = ROWS_PER_TILE // ZR
    rem = ROWS_PER_TILE - nfull * ZR
    for z in range(nfull):
        pltpu.sync_copy(zero_h, acc.at[pl.ds(r0 + z * ZR, ZR)])
    if rem:
        pltpu.sync_copy(zero_h.at[pl.ds(0, rem)],
                        acc.at[pl.ds(r0 + nfull * ZR, rem)])


def _counts_call(dst2d, int2d, bnd2d, basis, zeros):
    td = dst2d.shape[0] // (NC * NS)
    ti = int2d.shape[0] // (NC * NS)
    tb = bnd2d.shape[0] // (NC * NS)
    mesh = plsc.VectorSubcoreMesh(core_axis_name="c", subcore_axis_name="s")

    @functools.partial(
        pl.kernel,
        out_type=jax.ShapeDtypeStruct((NC, ACC_ROWS, 16), jnp.float32),
        mesh=mesh,
        compiler_params=pltpu.CompilerParams(use_tc_tiling_on_sc=False),
        scratch_types=[
            pltpu.VMEM_SHARED((ACC_ROWS, 16), jnp.float32),
            pltpu.VMEM((KJ, CH), jnp.int32),
            pltpu.VMEM((3, CH, 16), jnp.float32),
            pltpu.SemaphoreType.DMA,
        ],
    )
    def k(dst_h, int_h, bnd_h, basis_h, zero_h, out_h, acc, ibufa, bbuf, ssem):
        c = lax.axis_index("c")
        s = lax.axis_index("s")
        w = c * NS + s

        pltpu.sync_copy(basis_h, bbuf)
        _zero_acc(zero_h, acc, s)
        plsc.subcore_barrier()

        def scan_list(idx_h, rows_per_tile, which):
            base = w * rows_per_tile
            nblk = rows_per_tile // KJ

            def drain():
                for j in range(KJ):
                    pltpu.make_async_copy(
                        bbuf.at[which], acc.at[ibufa.at[j]], ssem).wait()

            def blk(i, _):
                @pl.when(i > 0)
                def _():
                    drain()

                pltpu.sync_copy(idx_h.at[pl.ds(base + i * KJ, KJ)], ibufa)
                for j in range(KJ):
                    pltpu.async_copy(bbuf.at[which], acc.at[ibufa.at[j]],
                                     ssem, add=True)
                return 0

            lax.fori_loop(0, nblk, blk, 0)
            drain()

        scan_list(dst_h, td, 0)
        scan_list(int_h, ti, 1)
        scan_list(bnd_h, tb, 2)

        plsc.subcore_barrier()
        r0 = s * ROWS_PER_TILE
        pltpu.sync_copy(acc.at[pl.ds(r0, ROWS_PER_TILE)],
                        out_h.at[c].at[pl.ds(r0, ROWS_PER_TILE)])

    return k(dst2d, int2d, bnd2d, basis, zeros)


def _scatter_call(comb2, gtab, zeros):
    spt = comb2.shape[0] // NS
    mesh = plsc.VectorSubcoreMesh(core_axis_name="c", subcore_axis_name="s")

    @functools.partial(
        pl.kernel,
        out_type=jax.ShapeDtypeStruct((NC, ACC_ROWS, 16), jnp.float32),
        mesh=mesh,
        compiler_params=pltpu.CompilerParams(use_tc_tiling_on_sc=False),
        scratch_types=[
            pltpu.VMEM_SHARED((ACC_ROWS, 16), jnp.float32),
            pltpu.VMEM((4 * KJ, CH), jnp.int32),
            pltpu.VMEM((KJ, CH, 16), jnp.float32),
            pltpu.SemaphoreType.DMA,
            pltpu.SemaphoreType.DMA,
        ],
    )
    def k(comb_h, gtab_h, zero_h, out_h, acc, ibuf, rows, gsem, ssem):
        c = lax.axis_index("c")
        s = lax.axis_index("s")

        _zero_acc(zero_h, acc, s)
        plsc.subcore_barrier()

        base = s * spt

        def drain_scatters(doff):
            for j in range(KJ):
                pltpu.make_async_copy(
                    rows.at[j], acc.at[ibuf.at[doff + j]], ssem).wait()

        def sub_block(soff, doff):
            descs = []
            for j in range(KJ):
                descs.append(
                    pltpu.async_copy(gtab_h.at[c].at[ibuf.at[soff + j]],
                                     rows.at[j], gsem))
            for j in range(KJ):
                descs[j].wait()
                pltpu.async_copy(rows.at[j], acc.at[ibuf.at[doff + j]],
                                 ssem, add=True)

        def blk(i, _):
            @pl.when(i > 0)
            def _():
                drain_scatters(3 * KJ)

            pltpu.sync_copy(comb_h.at[base + i], ibuf)
            sub_block(0, KJ)
            drain_scatters(KJ)
            sub_block(2 * KJ, 3 * KJ)
            return 0

        lax.fori_loop(0, spt, blk, 0)
        drain_scatters(3 * KJ)

        plsc.subcore_barrier()
        r0 = s * ROWS_PER_TILE
        pltpu.sync_copy(acc.at[pl.ds(r0, ROWS_PER_TILE)],
                        out_h.at[c].at[pl.ds(r0, ROWS_PER_TILE)])

    return k(comb2, gtab, zeros)


BR = 2000


def _full(shape):
    return pl.BlockSpec(shape, lambda i: (0,) * len(shape))


def _plane(p):
    return pl.BlockSpec((1, BR, 16), lambda i, p=p: (p, i, 0))


def _planes2():
    return pl.BlockSpec((2, BR, 16), lambda i: (0, i, 0))


def _rows(width):
    return pl.BlockSpec((BR, width), lambda i: (i, 0))


def _tc_entry_body(in8_r, c0_r, c1_r, wb1_r, bb1_r, wb2_r, bb2_r,
                   wi1_r, bi1_r, wi2_r, bi2_r, wc1_r,
                   g_r, dv_r):
    xin = in8_r[...]
    cnt = c0_r[0] + c1_r[0]
    dinv = lax.rsqrt(1.0 + cnt[:, 0:1])
    mi = cnt[:, 1:2] > 0.0
    mb = cnt[:, 2:3] > 0.0
    tb = jnp.maximum(xin @ wb1_r[...] + bb1_r[...], 0.0) @ wb2_r[...] + bb2_r[...]
    ti = jnp.maximum(xin @ wi1_r[...] + bi1_r[...], 0.0) @ wi2_r[...] + bi2_r[...]
    h0 = jnp.maximum(jnp.where(mi, ti, jnp.where(mb, tb, 0.0)), 0.0)
    g = (h0 @ wc1_r[...]) * dinv
    g_r[0] = g[:, :16]
    g_r[1] = g[:, 16:]
    dv_r[...] = dinv


def _tc_entry(in8, cnt, wb1, bb1, wb2, bb2, wi1, bi1, wi2, bi2, wc1):
    n = in8.shape[0]
    return pl.pallas_call(
        _tc_entry_body,
        grid=(n // BR,),
        in_specs=[_rows(8), _plane(0), _plane(1),
                  _full((8, 32)), _full((1, 32)), _full((32, 32)), _full((1, 32)),
                  _full((8, 32)), _full((1, 32)), _full((32, 32)), _full((1, 32)),
                  _full((32, 32))],
        out_specs=[_planes2(), _rows(1)],
        out_shape=[jax.ShapeDtypeStruct((2, n, 16), jnp.float32),
                   jax.ShapeDtypeStruct((n, 1), jnp.float32)],
    )(in8, cnt, cnt, wb1, bb1, wb2, bb2, wi1, bi1, wi2, bi2, wc1)


def _tc_mid_body(s0_r, s1_r, g_r, dv_r, b_r, wn_r, o_r):
    dinv = dv_r[...]
    sagg = jnp.concatenate([s0_r[0], s1_r[0]], axis=1)
    g = jnp.concatenate([g_r[0], g_r[1]], axis=1)
    h = jnp.maximum(dinv * (sagg + g) + b_r[...], 0.0)
    gn = (h @ wn_r[...]) * dinv
    o_r[0] = gn[:, :16]
    o_r[1] = gn[:, 16:]


def _tc_mid(s_acc, g, dv, b, wn):
    n = g.shape[1]
    return pl.pallas_call(
        _tc_mid_body,
        grid=(n // BR,),
        in_specs=[_plane(0), _plane(1), _planes2(), _rows(1),
                  _full((1, 32)), _full((32, 32))],
        out_specs=_planes2(),
        out_shape=jax.ShapeDtypeStruct((2, n, 16), jnp.float32),
    )(s_acc, s_acc, g, dv, b, wn)


def _tc_final_body(s0_r, s1_r, g_r, dv_r, b_r, wfc_r, bfc_r, o_r):
    dinv = dv_r[...]
    sagg = jnp.concatenate([s0_r[0], s1_r[0]], axis=1)
    g = jnp.concatenate([g_r[0], g_r[1]], axis=1)
    h = jnp.maximum(dinv * (sagg + g) + b_r[...], 0.0)
    o_r[...] = h @ wfc_r[...] + bfc_r[...]


def _tc_final(s_acc, g, dv, b, wfc, bfc):
    n = g.shape[1]
    return pl.pallas_call(
        _tc_final_body,
        grid=(n // BR,),
        in_specs=[_plane(0), _plane(1), _planes2(), _rows(1),
                  _full((1, 32)), _full((32, 1)), _full((1, 1))],
        out_specs=_rows(1),
        out_shape=jax.ShapeDtypeStruct((n, 1), jnp.float32),
    )(s_acc, s_acc, g, dv, b, wfc, bfc)


def kernel(x, y, edge_index, boundary_index, interior_index,
           W_b1, b_b1, W_b2, b_b2, W_i1, b_i1, W_i2, b_i2,
           Wc1, bc1, Wc2, bc2, Wc3, bc3, W_fc, b_fc):
    n = x.shape[0]
    e = edge_index.shape[1]
    src, dst = edge_index[0], edge_index[1]

    blkc = CH * KJ
    ep = _ceil_to(e, NS * blkc)
    ep = _ceil_to(ep, NC * NS * blkc)
    src_p = _pad_idx(src, ep, 0)
    dst_p = _pad_idx(dst, ep, n)
    int_p = _pad_idx(interior_index,
                     _ceil_to(interior_index.shape[0], NC * NS * blkc), n)
    bnd_p = _pad_idx(boundary_index,
                     _ceil_to(boundary_index.shape[0], NC * NS * blkc), n)

    basis = jnp.zeros((3, CH, 16), jnp.float32)
    basis = basis.at[0, :, 0].set(1.0).at[1, :, 1].set(1.0).at[2, :, 2].set(1.0)
    zeros = jnp.zeros((ZR, 16), jnp.float32)

    wb1 = jnp.pad(W_b1, ((0, 0), (0, 5))).T
    wi1 = jnp.pad(W_i1, ((0, 0), (0, 6))).T
    in8 = jnp.concatenate([x, y[:, None], jnp.zeros((n, 5), jnp.float32)],
                          axis=1)
    wc = [Wc1.T, Wc2.T, Wc3.T]
    bc = [bc1[None, :], bc2[None, :], bc3[None, :]]

    nb = src_p.shape[0] // KJ
    comb = jnp.concatenate([src_p.reshape(nb, KJ, CH),
                            dst_p.reshape(nb, KJ, CH)], axis=1)
    comb2 = comb.reshape(nb // 2, 4 * KJ, CH)

    cnt = _counts_call(dst_p, int_p, bnd_p, basis, zeros)

    g, dv = _tc_entry(in8, cnt, wb1, b_b1[None, :], W_b2.T, b_b2[None, :],
                      wi1, b_i1[None, :], W_i2.T, b_i2[None, :], wc[0])

    for layer in range(3):
        s_acc = _scatter_call(comb2, g, zeros)
        if layer < 2:
            g = _tc_mid(s_acc, g, dv, bc[layer], wc[layer + 1])
        else:
            out = _tc_final(s_acc, g, dv, bc[layer], W_fc.T, b_fc[None, :])
    return out
